# baseline (speedup 1.0000x reference)
import functools

import jax
import jax.numpy as jnp
from jax import lax
from jax.experimental import pallas as pl
from jax.experimental.pallas import tpu as pltpu
from jax.experimental.pallas import tpu_sc as plsc

N = 10000
E = 320000
D = 128

NC = 2
NS = 16
NW = NC * NS

N_PAD = 10000
RT = 640
RT_LAST = N_PAD - 15 * RT
N_DEG = 10240

C = 80
EW = E // NW
ITERS = EW // C
BLOCKS = 5
IT_B = ITERS // BLOCKS

_mesh = plsc.VectorSubcoreMesh(core_axis_name="c", subcore_axis_name="s")

_f32 = jnp.float32


def _zero_vec_ref(ref, n):
    z = jnp.zeros((16,), _f32)
    def body(i, _):
        ref[pl.ds(i * 16, 16)] = z
        return _
    lax.fori_loop(0, n // 16, body, None)


def _deg_body(dst_hbm, out_hbm, deg_sp, zbuf, ones_v, idx2d, semA, semB):
    tid = lax.axis_index("s")
    cid = lax.axis_index("c")
    wid = tid * NC + cid

    _zero_vec_ref(zbuf, RT)
    one = jnp.ones((16,), _f32)
    for i in range(C // 16):
        ones_v[pl.ds(i * 16, 16)] = one
    pltpu.sync_copy(zbuf, deg_sp.at[pl.ds(tid * RT, RT)])
    plsc.subcore_barrier()

    def scat(j, sem):
        return pltpu.async_copy(ones_v, deg_sp.at[idx2d.at[j]], sem, add=True)

    def swait(j, sem):
        pltpu.make_async_copy(ones_v, deg_sp.at[idx2d.at[j]], sem).wait()

    for blk in range(BLOCKS):
        pltpu.sync_copy(dst_hbm.at[wid, blk], idx2d)
        scat(0, semA)
        def dbody(i, _):
            j = 2 * i
            scat(j + 1, semB)
            swait(j, semA)
            scat(j + 2, semA)
            swait(j + 1, semB)
            return _
        lax.fori_loop(0, (IT_B - 1) // 2, dbody, None)
        swait(IT_B - 1, semA)

    plsc.subcore_barrier()
    pltpu.sync_copy(deg_sp.at[pl.ds(tid * RT, RT)],
                    out_hbm.at[cid, pl.ds(tid * RT, RT)])


@jax.jit
def _sc_degree(dst2d):
    return pl.kernel(
        _deg_body,
        out_type=jax.ShapeDtypeStruct((NC, N_DEG), _f32),
        mesh=_mesh,
        scratch_types=[
            pltpu.VMEM_SHARED((N_DEG,), _f32),
            pltpu.VMEM((640,), _f32),
            pltpu.VMEM((C,), _f32),
            pltpu.VMEM((IT_B, C), jnp.int32),
            pltpu.SemaphoreType.DMA,
            pltpu.SemaphoreType.DMA,
        ],
    )(dst2d)


def _agg_body(g_hbm, src_hbm, dst_hbm, out_hbm, acc, sidx2d, didx2d,
              rowsA, rowsB, rowsC, rowsD,
              gsemA, gsemB, gsemC, gsemD, ssemA, ssemB, ssemC, ssemD):
    tid = lax.axis_index("s")
    cid = lax.axis_index("c")
    wid = tid * NC + cid

    def zr(r, _):
        z = jnp.zeros((16,), _f32)
        for j in range(D // 16):
            rowsA[r, pl.ds(j * 16, 16)] = z
        return _
    lax.fori_loop(0, C, zr, None)
    for k in range(RT_LAST // C):
        pltpu.sync_copy(rowsA, acc.at[pl.ds(tid * RT + k * C, C)])
    @pl.when(tid < NS - 1)
    def _():
        for k in range(RT_LAST // C, RT // C):
            pltpu.sync_copy(rowsA, acc.at[pl.ds(tid * RT + k * C, C)])
    plsc.subcore_barrier()

    cur_idx = [None, None]

    def gat(j, buf, sem):
        return pltpu.async_copy(g_hbm.at[cur_idx[0].at[j]], buf, sem)

    def gwait(j, buf, sem):
        pltpu.make_async_copy(g_hbm.at[cur_idx[0].at[j]], buf, sem).wait()

    def scat(j, buf, sem):
        return pltpu.async_copy(buf, acc.at[cur_idx[1].at[j]], sem, add=True)

    def swait(j, buf, sem):
        pltpu.make_async_copy(buf, acc.at[cur_idx[1].at[j]], sem).wait()

    bufs = (rowsA, rowsB, rowsC, rowsD)
    gsems = (gsemA, gsemB, gsemC, gsemD)
    ssems = (ssemA, ssemB, ssemC, ssemD)

    def step(j, b, jm1, jp3, last):
        gwait(j, bufs[b], gsems[b])
        scat(j, bufs[b], ssems[b])
        if jm1 is not None:
            bm = (b + 3) % 4
            swait(jm1, bufs[bm], ssems[bm])
        if jp3 is not None:
            bp = (b + 3) % 4
            gat(jp3, bufs[bp], gsems[bp])
        if last:
            swait(j, bufs[b], ssems[b])

    for blk in range(BLOCKS):
        cur_idx[0], cur_idx[1] = sidx2d, didx2d
        pltpu.sync_copy(src_hbm.at[wid, blk], sidx2d)
        pltpu.sync_copy(dst_hbm.at[wid, blk], didx2d)
        gat(0, bufs[0], gsems[0])
        gat(1, bufs[1], gsems[1])
        gat(2, bufs[2], gsems[2])
        step(0, 0, None, 3, False)
        step(1, 1, 0, 4, False)
        step(2, 2, 1, 5, False)
        step(3, 3, 2, 6, False)
        def body(i, _):
            j0 = 4 * i
            step(j0, 0, j0 - 1, j0 + 3, False)
            step(j0 + 1, 1, j0, j0 + 4, False)
            step(j0 + 2, 2, j0 + 1, j0 + 5, False)
            step(j0 + 3, 3, j0 + 2, j0 + 6, False)
            return _
        lax.fori_loop(1, (IT_B - 5) // 4, body, None)
        step(IT_B - 5, 0, IT_B - 6, IT_B - 2, False)
        step(IT_B - 4, 1, IT_B - 5, IT_B - 1, False)
        step(IT_B - 3, 2, IT_B - 4, None, False)
        step(IT_B - 2, 3, IT_B - 3, None, False)
        step(IT_B - 1, 0, IT_B - 2, None, True)

    plsc.subcore_barrier()
    @pl.when(tid < NS - 1)
    def _():
        pltpu.sync_copy(acc.at[pl.ds(tid * RT, RT)],
                        out_hbm.at[cid, pl.ds(tid * RT, RT)])
    @pl.when(tid == NS - 1)
    def _():
        pltpu.sync_copy(acc.at[pl.ds(tid * RT, RT_LAST)],
                        out_hbm.at[cid, pl.ds(tid * RT, RT_LAST)])


@jax.jit
def _sc_aggregate(g, src2d, dst2d):
    return pl.kernel(
        _agg_body,
        out_type=jax.ShapeDtypeStruct((NC, N_PAD, D), _f32),
        mesh=_mesh,
        scratch_types=[
            pltpu.VMEM_SHARED((N_PAD, D), _f32),
            pltpu.VMEM((IT_B, C), jnp.int32),
            pltpu.VMEM((IT_B, C), jnp.int32),
            pltpu.VMEM((C, D), _f32),
            pltpu.VMEM((C, D), _f32),
            pltpu.VMEM((C, D), _f32),
            pltpu.VMEM((C, D), _f32),
            pltpu.SemaphoreType.DMA,
            pltpu.SemaphoreType.DMA,
            pltpu.SemaphoreType.DMA,
            pltpu.SemaphoreType.DMA,
            pltpu.SemaphoreType.DMA,
            pltpu.SemaphoreType.DMA,
            pltpu.SemaphoreType.DMA,
            pltpu.SemaphoreType.DMA,
        ],
    )(g, src2d, dst2d)


_BM = 1000


def _tc1_body(x_ref, w_ref, dp_ref, g_ref, dinv_ref):
    deg = dp_ref[0] + dp_ref[1] + 1.0
    dv = lax.rsqrt(deg)
    h = jnp.dot(x_ref[...], w_ref[...], preferred_element_type=_f32)
    g_ref[...] = h * dv
    dinv_ref[...] = dv


@jax.jit
def _tc_stage1(x, W1, deg_parts):
    grid = (N // _BM,)
    return pl.pallas_call(
        _tc1_body,
        grid=grid,
        in_specs=[
            pl.BlockSpec((_BM, D), lambda i: (i, 0)),
            pl.BlockSpec((D, D), lambda i: (0, 0)),
            pl.BlockSpec((NC, _BM, 1), lambda i: (0, i, 0)),
        ],
        out_specs=[
            pl.BlockSpec((_BM, D), lambda i: (i, 0)),
            pl.BlockSpec((_BM, 1), lambda i: (i, 0)),
        ],
        out_shape=[
            jax.ShapeDtypeStruct((N, D), _f32),
            jax.ShapeDtypeStruct((N, 1), _f32),
        ],
    )(x, W1, deg_parts)


def _tc2_body(a_ref, g_ref, dinv_ref, b_ref, w_ref, out_ref):
    dv = dinv_ref[...]
    z = dv * (a_ref[0] + a_ref[1] + g_ref[...]) + b_ref[...]
    z = jnp.maximum(z, 0.0)
    out_ref[...] = jnp.dot(z, w_ref[...], preferred_element_type=_f32) * dv


@jax.jit
def _tc_stage2(agg, g1, dinv, b1, W2):
    grid = (N // _BM,)
    return pl.pallas_call(
        _tc2_body,
        grid=grid,
        in_specs=[
            pl.BlockSpec((NC, _BM, D), lambda i: (0, i, 0)),
            pl.BlockSpec((_BM, D), lambda i: (i, 0)),
            pl.BlockSpec((_BM, 1), lambda i: (i, 0)),
            pl.BlockSpec((1, D), lambda i: (0, 0)),
            pl.BlockSpec((D, D), lambda i: (0, 0)),
        ],
        out_specs=pl.BlockSpec((_BM, D), lambda i: (i, 0)),
        out_shape=jax.ShapeDtypeStruct((N, D), _f32),
    )(agg, g1, dinv, b1, W2)


def _tc3_body(a_ref, g_ref, dinv_ref, b_ref, out_ref):
    out_ref[...] = (dinv_ref[...] * (a_ref[0] + a_ref[1] + g_ref[...])
                    + b_ref[...])


@jax.jit
def _tc_stage3(agg, g2, dinv, b2):
    grid = (N // _BM,)
    return pl.pallas_call(
        _tc3_body,
        grid=grid,
        in_specs=[
            pl.BlockSpec((NC, _BM, D), lambda i: (0, i, 0)),
            pl.BlockSpec((_BM, D), lambda i: (i, 0)),
            pl.BlockSpec((_BM, 1), lambda i: (i, 0)),
            pl.BlockSpec((1, D), lambda i: (0, 0)),
        ],
        out_specs=pl.BlockSpec((_BM, D), lambda i: (i, 0)),
        out_shape=jax.ShapeDtypeStruct((N, D), _f32),
    )(agg, g2, dinv, b2)


def kernel(x, edge_index, W1, b1, W2, b2):
    src = edge_index[0].astype(jnp.int32).reshape(NW, BLOCKS, IT_B, C)
    dst = edge_index[1].astype(jnp.int32).reshape(NW, BLOCKS, IT_B, C)

    deg_parts = _sc_degree(dst).reshape(NC, N_DEG, 1)
    g1, dinv = _tc_stage1(x, W1, deg_parts)

    agg1 = _sc_aggregate(g1, src, dst)
    g2 = _tc_stage2(agg1, g1, dinv, b1.reshape(1, D), W2)

    agg2 = _sc_aggregate(g2, src, dst)
    out = _tc_stage3(agg2, g2, dinv, b2.reshape(1, D))
    return out


# speedup vs baseline: 1.0480x; 1.0480x over previous
import functools

import jax
import jax.numpy as jnp
from jax import lax
from jax.experimental import pallas as pl
from jax.experimental.pallas import tpu as pltpu
from jax.experimental.pallas import tpu_sc as plsc

N = 10000
E = 320000
D = 128

NC = 2
NS = 16
NW = NC * NS

N_PAD = 10240
RT = N_PAD // NS

C = 80
EW = E // NW
ITERS = EW // C
BLOCKS = 5
IT_B = ITERS // BLOCKS

_mesh = plsc.VectorSubcoreMesh(core_axis_name="c", subcore_axis_name="s")

_f32 = jnp.float32


def _zero_vec_ref(ref, n):
    z = jnp.zeros((16,), _f32)
    def body(i, _):
        ref[pl.ds(i * 16, 16)] = z
        return _
    lax.fori_loop(0, n // 16, body, None)


def _deg_body(dst_hbm, out_hbm, deg_sp, zbuf, ones_v, idx2d, semA, semB):
    tid = lax.axis_index("s")
    cid = lax.axis_index("c")
    wid = tid * NC + cid

    _zero_vec_ref(zbuf, RT)
    one = jnp.ones((16,), _f32)
    for i in range(C // 16):
        ones_v[pl.ds(i * 16, 16)] = one
    pltpu.sync_copy(zbuf, deg_sp.at[pl.ds(tid * RT, RT)])
    plsc.subcore_barrier()

    def scat(j, sem):
        return pltpu.async_copy(ones_v, deg_sp.at[idx2d.at[j]], sem, add=True)

    def swait(j, sem):
        pltpu.make_async_copy(ones_v, deg_sp.at[idx2d.at[j]], sem).wait()

    for blk in range(BLOCKS):
        pltpu.sync_copy(dst_hbm.at[wid, blk], idx2d)
        scat(0, semA)
        def body(i, _):
            j = 2 * i
            scat(j + 1, semB)
            swait(j, semA)
            scat(j + 2, semA)
            swait(j + 1, semB)
            return _
        lax.fori_loop(0, (IT_B - 1) // 2, body, None)
        swait(IT_B - 1, semA)

    plsc.subcore_barrier()
    pltpu.sync_copy(deg_sp.at[pl.ds(tid * RT, RT)],
                    out_hbm.at[cid, pl.ds(tid * RT, RT)])


@jax.jit
def _sc_degree(dst2d):
    return pl.kernel(
        _deg_body,
        out_type=jax.ShapeDtypeStruct((NC, N_PAD), _f32),
        mesh=_mesh,
        scratch_types=[
            pltpu.VMEM_SHARED((N_PAD,), _f32),
            pltpu.VMEM((RT,), _f32),
            pltpu.VMEM((C,), _f32),
            pltpu.VMEM((IT_B, C), jnp.int32),
            pltpu.SemaphoreType.DMA,
            pltpu.SemaphoreType.DMA,
        ],
    )(dst2d)


def _agg_body(g_hbm, src_hbm, dst_hbm, out_hbm, acc, sidx2d, didx2d,
              sidx2d2, didx2d2, rowsA, rowsB, rowsC,
              gsemA, gsemB, gsemC, ssemA, ssemB, ssemC, isemS, isemD):
    tid = lax.axis_index("s")
    cid = lax.axis_index("c")
    wid = tid * NC + cid

    def zr(r, _):
        z = jnp.zeros((16,), _f32)
        for j in range(D // 16):
            rowsA[r, pl.ds(j * 16, 16)] = z
        return _
    lax.fori_loop(0, C, zr, None)
    for k in range(RT // C):
        pltpu.sync_copy(rowsA, acc.at[pl.ds(tid * RT + k * C, C)])
    plsc.subcore_barrier()

    cur_idx = [None, None]

    def gat(j, buf, sem):
        return pltpu.async_copy(g_hbm.at[cur_idx[0].at[j]], buf, sem)

    def gwait(j, buf, sem):
        pltpu.make_async_copy(g_hbm.at[cur_idx[0].at[j]], buf, sem).wait()

    def scat(j, buf, sem):
        return pltpu.async_copy(buf, acc.at[cur_idx[1].at[j]], sem, add=True)

    def swait(j, buf, sem):
        pltpu.make_async_copy(buf, acc.at[cur_idx[1].at[j]], sem).wait()

    bufs = (rowsA, rowsB, rowsC)
    gsems = (gsemA, gsemB, gsemC)
    ssems = (ssemA, ssemB, ssemC)
    sidxs = (sidx2d, sidx2d2)
    didxs = (didx2d, didx2d2)

    def step(j, b, jm1, jp2, last):
        gwait(j, bufs[b], gsems[b])
        scat(j, bufs[b], ssems[b])
        if jm1 is not None:
            bm = (b + 2) % 3
            swait(jm1, bufs[bm], ssems[bm])
        if jp2 is not None:
            bp = (b + 2) % 3
            gat(jp2, bufs[bp], gsems[bp])
        if last:
            swait(j, bufs[b], ssems[b])

    pltpu.sync_copy(src_hbm.at[wid, 0], sidxs[0])
    pltpu.sync_copy(dst_hbm.at[wid, 0], didxs[0])
    for blk in range(BLOCKS):
        cur, nxt = blk % 2, (blk + 1) % 2
        cur_idx[0], cur_idx[1] = sidxs[cur], didxs[cur]
        gat(0, bufs[0], gsems[0])
        gat(1, bufs[1], gsems[1])
        if blk + 1 < BLOCKS:
            pltpu.async_copy(src_hbm.at[wid, blk + 1], sidxs[nxt], isemS)
            pltpu.async_copy(dst_hbm.at[wid, blk + 1], didxs[nxt], isemD)
        step(0, 0, None, 2, False)
        step(1, 1, 0, 3, False)
        step(2, 2, 1, 4, False)
        def body(i, _):
            j0 = 3 * i
            step(j0, 0, j0 - 1, j0 + 2, False)
            step(j0 + 1, 1, j0, j0 + 3, False)
            step(j0 + 2, 2, j0 + 1, j0 + 4, False)
            return _
        lax.fori_loop(1, (IT_B - 4) // 3, body, None)
        step(IT_B - 4, 0, IT_B - 5, IT_B - 2, False)
        step(IT_B - 3, 1, IT_B - 4, IT_B - 1, False)
        step(IT_B - 2, 2, IT_B - 3, None, False)
        step(IT_B - 1, 0, IT_B - 2, None, True)
        if blk + 1 < BLOCKS:
            pltpu.make_async_copy(src_hbm.at[wid, blk + 1], sidxs[nxt], isemS).wait()
            pltpu.make_async_copy(dst_hbm.at[wid, blk + 1], didxs[nxt], isemD).wait()

    plsc.subcore_barrier()
    pltpu.sync_copy(acc.at[pl.ds(tid * RT, RT)],
                    out_hbm.at[cid, pl.ds(tid * RT, RT)])


@jax.jit
def _sc_aggregate(g, src2d, dst2d):
    return pl.kernel(
        _agg_body,
        out_type=jax.ShapeDtypeStruct((NC, N_PAD, D), _f32),
        mesh=_mesh,
        scratch_types=[
            pltpu.VMEM_SHARED((N_PAD, D), _f32),
            pltpu.VMEM((IT_B, C), jnp.int32),
            pltpu.VMEM((IT_B, C), jnp.int32),
            pltpu.VMEM((IT_B, C), jnp.int32),
            pltpu.VMEM((IT_B, C), jnp.int32),
            pltpu.VMEM((C, D), _f32),
            pltpu.VMEM((C, D), _f32),
            pltpu.VMEM((C, D), _f32),
            pltpu.SemaphoreType.DMA,
            pltpu.SemaphoreType.DMA,
            pltpu.SemaphoreType.DMA,
            pltpu.SemaphoreType.DMA,
            pltpu.SemaphoreType.DMA,
            pltpu.SemaphoreType.DMA,
            pltpu.SemaphoreType.DMA,
            pltpu.SemaphoreType.DMA,
        ],
    )(g, src2d, dst2d)


_BM = 1000


def _tc1_body(x_ref, w_ref, dp_ref, g_ref, dinv_ref):
    deg = dp_ref[0] + dp_ref[1] + 1.0
    dv = lax.rsqrt(deg)
    h = jnp.dot(x_ref[...], w_ref[...], preferred_element_type=_f32)
    g_ref[...] = h * dv
    dinv_ref[...] = dv


@jax.jit
def _tc_stage1(x, W1, deg_parts):
    grid = (N // _BM,)
    return pl.pallas_call(
        _tc1_body,
        grid=grid,
        in_specs=[
            pl.BlockSpec((_BM, D), lambda i: (i, 0)),
            pl.BlockSpec((D, D), lambda i: (0, 0)),
            pl.BlockSpec((NC, _BM, 1), lambda i: (0, i, 0)),
        ],
        out_specs=[
            pl.BlockSpec((_BM, D), lambda i: (i, 0)),
            pl.BlockSpec((_BM, 1), lambda i: (i, 0)),
        ],
        out_shape=[
            jax.ShapeDtypeStruct((N, D), _f32),
            jax.ShapeDtypeStruct((N, 1), _f32),
        ],
    )(x, W1, deg_parts)


def _tc2_body(a_ref, g_ref, dinv_ref, b_ref, w_ref, out_ref):
    dv = dinv_ref[...]
    z = dv * (a_ref[0] + a_ref[1] + g_ref[...]) + b_ref[...]
    z = jnp.maximum(z, 0.0)
    out_ref[...] = jnp.dot(z, w_ref[...], preferred_element_type=_f32) * dv


@jax.jit
def _tc_stage2(agg, g1, dinv, b1, W2):
    grid = (N // _BM,)
    return pl.pallas_call(
        _tc2_body,
        grid=grid,
        in_specs=[
            pl.BlockSpec((NC, _BM, D), lambda i: (0, i, 0)),
            pl.BlockSpec((_BM, D), lambda i: (i, 0)),
            pl.BlockSpec((_BM, 1), lambda i: (i, 0)),
            pl.BlockSpec((1, D), lambda i: (0, 0)),
            pl.BlockSpec((D, D), lambda i: (0, 0)),
        ],
        out_specs=pl.BlockSpec((_BM, D), lambda i: (i, 0)),
        out_shape=jax.ShapeDtypeStruct((N, D), _f32),
    )(agg, g1, dinv, b1, W2)


def _tc3_body(a_ref, g_ref, dinv_ref, b_ref, out_ref):
    out_ref[...] = (dinv_ref[...] * (a_ref[0] + a_ref[1] + g_ref[...])
                    + b_ref[...])


@jax.jit
def _tc_stage3(agg, g2, dinv, b2):
    grid = (N // _BM,)
    return pl.pallas_call(
        _tc3_body,
        grid=grid,
        in_specs=[
            pl.BlockSpec((NC, _BM, D), lambda i: (0, i, 0)),
            pl.BlockSpec((_BM, D), lambda i: (i, 0)),
            pl.BlockSpec((_BM, 1), lambda i: (i, 0)),
            pl.BlockSpec((1, D), lambda i: (0, 0)),
        ],
        out_specs=pl.BlockSpec((_BM, D), lambda i: (i, 0)),
        out_shape=jax.ShapeDtypeStruct((N, D), _f32),
    )(agg, g2, dinv, b2)


def kernel(x, edge_index, W1, b1, W2, b2):
    src = edge_index[0].astype(jnp.int32).reshape(NW, BLOCKS, IT_B, C)
    dst = edge_index[1].astype(jnp.int32).reshape(NW, BLOCKS, IT_B, C)

    deg_parts = _sc_degree(dst).reshape(NC, N_PAD, 1)
    g1, dinv = _tc_stage1(x, W1, deg_parts)

    agg1 = _sc_aggregate(g1, src, dst)
    g2 = _tc_stage2(agg1, g1, dinv, b1.reshape(1, D), W2)

    agg2 = _sc_aggregate(g2, src, dst)
    out = _tc_stage3(agg2, g2, dinv, b2.reshape(1, D))
    return out
